# baseline (speedup 1.0000x reference)
import jax
import jax.numpy as jnp
from jax.experimental import pallas as pl
from jax.experimental.pallas import tpu as pltpu

GI = 5
STEPS = 25 // GI


def _body(bp_ref, rt_ref, rw_ref, out_ref, acc_ref):
    step = pl.program_id(0)

    @pl.when(step == 0)
    def _():
        acc_ref[0] = 0.0
        acc_ref[1] = 0.0

    p = bp_ref[...].reshape(GI * 25, 128)
    t = rt_ref[...].reshape(GI * 25, 128)
    w = rw_ref[...]

    def phase_sum(x):
        return x + jnp.roll(x, -64, axis=1)

    sp = phase_sum(p)
    st = phase_sum(t)
    sm = phase_sum(jnp.minimum(p, t))
    p_area = sp * jnp.roll(sp, -32, axis=1)
    t_area = st * jnp.roll(st, -32, axis=1)
    a_i = sm * jnp.roll(sm, -32, axis=1)
    a_u = t_area + p_area - a_i
    ratio = (a_i + 1.0) / jnp.maximum(a_u + 1.0, 1e-6)
    lg = jnp.log(jnp.maximum(ratio, 1e-30))

    lg0 = lg.reshape(GI, 25, 128)[:, :, :32]
    m = (w > 0.0).astype(jnp.float32)
    m_t = jnp.transpose(m, (0, 2, 1))
    acc_ref[0] = acc_ref[0] + jnp.sum(lg0 * m_t)
    acc_ref[1] = acc_ref[1] + jnp.sum(m)

    @pl.when(step == pl.num_programs(0) - 1)
    def _():
        out_ref[0, 0] = -acc_ref[0] / jnp.maximum(acc_ref[1], 1.0)


@jax.jit
def _iou_loss(bpt, rtt, rwj):
    return pl.pallas_call(
        _body,
        grid=(STEPS,),
        in_specs=[
            pl.BlockSpec((GI, 25, 4, 32), lambda i: (i, 0, 0, 0)),
            pl.BlockSpec((GI, 25, 4, 32), lambda i: (i, 0, 0, 0)),
            pl.BlockSpec((GI, 32, 25), lambda i: (i, 0, 0)),
        ],
        out_shape=jax.ShapeDtypeStruct((1, 1), jnp.float32),
        out_specs=pl.BlockSpec(
            (1, 1), lambda i: (0, 0), memory_space=pltpu.SMEM
        ),
        scratch_shapes=[pltpu.SMEM((2,), jnp.float32)],
    )(bpt, rtt, rwj)


def kernel(bbox_pred, reg_target, reg_weight):
    bpt = jnp.transpose(bbox_pred, (2, 3, 1, 0))
    rtt = jnp.transpose(reg_target, (1, 2, 3, 0))
    rwj = jnp.transpose(reg_weight, (1, 0, 2))
    return _iou_loss(bpt, rtt, rwj)[0, 0]


# speedup vs baseline: 1.3452x; 1.3452x over previous
import jax
import jax.numpy as jnp
from jax import lax
from jax.experimental import pallas as pl
from jax.experimental.pallas import tpu as pltpu


def _body(bp_ref, rt_ref, rw_ref, out_ref):
    p = bp_ref[...].reshape(625, 128)
    t = rt_ref[...].reshape(625, 128)
    w = rw_ref[...]

    def phase_sum(x):
        return x + jnp.roll(x, -64, axis=1)

    sp = phase_sum(p)
    st = phase_sum(t)
    sm = phase_sum(jnp.minimum(p, t))
    p_area = sp * jnp.roll(sp, -32, axis=1)
    t_area = st * jnp.roll(st, -32, axis=1)
    a_i = sm * jnp.roll(sm, -32, axis=1)
    a_u = t_area + p_area - a_i
    lg = jnp.log((a_i + 1.0) / (a_u + 1.0))

    lg0 = lg.reshape(25, 25, 128)[:, :, :32]
    m = (w > 0.0).astype(jnp.float32)
    m_t = jnp.transpose(m, (0, 2, 1))
    s = jnp.sum(lg0 * m_t)
    c = jnp.sum(m)
    out_ref[0, 0] = -s / jnp.maximum(c, 1.0)


@jax.jit
def _iou_loss(bpt, rtt, rwj):
    return pl.pallas_call(
        _body,
        out_shape=jax.ShapeDtypeStruct((1, 1), jnp.float32),
        out_specs=pl.BlockSpec(memory_space=pltpu.SMEM),
    )(bpt, rtt, rwj)


def kernel(bbox_pred, reg_target, reg_weight):
    bpt = jnp.transpose(bbox_pred, (2, 3, 1, 0))
    rtt = jnp.transpose(reg_target, (1, 2, 3, 0))
    rwj = jnp.transpose(reg_weight, (1, 0, 2))
    return _iou_loss(bpt, rtt, rwj)[0, 0]
